# baseline (speedup 1.0000x reference)
import functools

import jax
import jax.numpy as jnp
from jax import lax
from jax.experimental import pallas as pl
from jax.experimental.pallas import tpu as pltpu
from jax.experimental.pallas import tpu_sc as plsc

E = 8
K = 2
D = 1024
FF = 4096
T = 2048
A = T * K
BT = 512
NBLK = A // BT + E
P = NBLK * BT
BF = 1024
NF = FF // BF
EPAD = 128


def _router_body(x_ref, gw_ref, logits_ref, route_ref):
    x = x_ref[...]
    gw = gw_ref[...]
    logits = jnp.dot(x, gw, preferred_element_type=jnp.float32)
    logits_ref[...] = logits
    col = lax.broadcasted_iota(jnp.int32, (T, EPAD), 1)
    valid = col < E
    ml = jnp.where(valid, logits, jnp.float32(-1e30))
    m = jnp.max(ml, axis=1, keepdims=True)
    ex = jnp.where(valid, jnp.exp(ml - m), 0.0)
    p = ex / jnp.sum(ex, axis=1, keepdims=True)
    w1 = jnp.max(p, axis=1, keepdims=True)
    e1 = jnp.min(jnp.where((p == w1) & valid, col, EPAD), axis=1, keepdims=True)
    p2 = jnp.where(valid & (col != e1), p, jnp.float32(-1.0))
    w2 = jnp.max(p2, axis=1, keepdims=True)
    e2 = jnp.min(jnp.where((p2 == w2) & valid, col, EPAD), axis=1, keepdims=True)
    s = w1 + w2
    w1n = w1 / s
    w2n = w2 / s
    route = jnp.where(col == 0, e1.astype(jnp.float32),
            jnp.where(col == 1, e2.astype(jnp.float32),
            jnp.where(col == 2, w1n,
            jnp.where(col == 3, w2n, 0.0))))
    route_ref[...] = route


def _router(x2d, gwt_pad):
    return pl.pallas_call(
        _router_body,
        out_shape=(jax.ShapeDtypeStruct((T, EPAD), jnp.float32),
                   jax.ShapeDtypeStruct((T, EPAD), jnp.float32)),
    )(x2d, gwt_pad)


def _make_sc_gather(B):
    info = plsc.get_sparse_core_info()
    nw = info.num_cores * info.num_subcores
    b_per_w = B // nw
    ch = min(b_per_w, 32)
    nch = b_per_w // ch
    nb = min(3, nch)
    mesh = plsc.VectorSubcoreMesh(core_axis_name="c", subcore_axis_name="s")

    @functools.partial(
        pl.kernel, mesh=mesh,
        out_type=jax.ShapeDtypeStruct((B, D), jnp.float32),
        scratch_types=[
            pltpu.VMEM((b_per_w,), jnp.int32),
            pltpu.VMEM((nb * ch, D), jnp.float32),
        ] + [pltpu.SemaphoreType.DMA] * (2 * 3),
    )
    def k(table_hbm, idx_hbm, out_hbm, idx_v, rows_v, *sems):
        gs, ws = sems[:3], sems[3:]
        wid = lax.axis_index("s") * info.num_cores + lax.axis_index("c")
        base = wid * b_per_w
        pltpu.sync_copy(idx_hbm.at[pl.ds(base, b_per_w)], idx_v)
        gcp = [None] * nb
        wcp = [None] * nb
        for c in range(min(nb, nch)):
            gcp[c] = pltpu.async_copy(
                table_hbm.at[idx_v.at[pl.ds(c * ch, ch)]],
                rows_v.at[pl.ds(c * ch, ch)], gs[c])
        for c in range(nch):
            b = c % nb
            gcp[b].wait()
            wcp[b] = pltpu.async_copy(
                rows_v.at[pl.ds(b * ch, ch)],
                out_hbm.at[pl.ds(base + c * ch, ch)], ws[b])
            nxt = c + nb
            if nxt < nch:
                wcp[b].wait()
                gcp[b] = pltpu.async_copy(
                    table_hbm.at[idx_v.at[pl.ds(nxt * ch, ch)]],
                    rows_v.at[pl.ds(b * ch, ch)], gs[b])
        for b in range(nb):
            if wcp[b] is not None:
                wcp[b].wait()

    return k


def _make_sc_combine():
    info = plsc.get_sparse_core_info()
    nw = info.num_cores * info.num_subcores
    t_per_w = T // nw
    ch = 32
    nch = t_per_w // ch
    mesh = plsc.VectorSubcoreMesh(core_axis_name="c", subcore_axis_name="s")

    @functools.partial(
        pl.kernel, mesh=mesh,
        out_type=jax.ShapeDtypeStruct((T, D), jnp.float32),
        scratch_types=[
            pltpu.VMEM((t_per_w,), jnp.int32),
            pltpu.VMEM((t_per_w,), jnp.int32),
            pltpu.VMEM((ch, D), jnp.float32),
            pltpu.VMEM((ch, D), jnp.float32),
            pltpu.SemaphoreType.DMA,
        ],
    )
    def k(table_hbm, p0_hbm, p1_hbm, out_hbm, i0_v, i1_v, r0_v, r1_v, sem):
        wid = lax.axis_index("s") * info.num_cores + lax.axis_index("c")
        base = wid * t_per_w
        pltpu.sync_copy(p0_hbm.at[pl.ds(base, t_per_w)], i0_v)
        pltpu.sync_copy(p1_hbm.at[pl.ds(base, t_per_w)], i1_v)
        for c in range(nch):
            pltpu.async_copy(
                table_hbm.at[i0_v.at[pl.ds(c * ch, ch)]], r0_v, sem).wait()
            pltpu.async_copy(
                table_hbm.at[i1_v.at[pl.ds(c * ch, ch)]], r1_v, sem).wait()

            def body(i, carry):
                for j in range(D // 16):
                    sl = pl.ds(j * 16, 16)
                    r0_v[i, sl] = r0_v[i, sl] + r1_v[i, sl]
                return carry

            lax.fori_loop(0, ch, body, 0)
            pltpu.sync_copy(r0_v, out_hbm.at[pl.ds(base + c * ch, ch)])

    return k


def _mlp_body(be_ref, bv_ref, x_ref, win_ref, bin_ref, wout_ref, bout_ref,
              rw_ref, out_ref):
    f = pl.program_id(1)
    b = pl.program_id(0)

    @pl.when(bv_ref[b] == 1)
    def _():
        x = x_ref[...].astype(jnp.bfloat16)
        win = win_ref[0].astype(jnp.bfloat16)
        h = jnp.dot(x, win, preferred_element_type=jnp.float32)
        h = h + bin_ref[0]
        a = jax.nn.gelu(h).astype(jnp.bfloat16)
        wout = wout_ref[0].astype(jnp.bfloat16)
        contrib = jnp.dot(a, wout, preferred_element_type=jnp.float32)

        @pl.when(f == 0)
        def _():
            out_ref[...] = contrib + bout_ref[0]

        @pl.when(f != 0)
        def _():
            out_ref[...] = out_ref[...] + contrib

        @pl.when(f == NF - 1)
        def _():
            w = rw_ref[...][:, 0:1]
            out_ref[...] = out_ref[...] * w


def _grouped_mlp(block_expert, block_valid, hs, W_in, b_in, W_out, b_out, rw2d):
    grid_spec = pltpu.PrefetchScalarGridSpec(
        num_scalar_prefetch=2,
        grid=(NBLK, NF),
        in_specs=[
            pl.BlockSpec((BT, D), lambda b, f, be, bv: (b, 0)),
            pl.BlockSpec((1, D, BF), lambda b, f, be, bv: (be[b], 0, f)),
            pl.BlockSpec((1, 1, BF), lambda b, f, be, bv: (be[b] * NF + f, 0, 0)),
            pl.BlockSpec((1, BF, D), lambda b, f, be, bv: (be[b], f, 0)),
            pl.BlockSpec((1, 1, D), lambda b, f, be, bv: (be[b], 0, 0)),
            pl.BlockSpec((BT, 128), lambda b, f, be, bv: (b, 0)),
        ],
        out_specs=pl.BlockSpec((BT, D), lambda b, f, be, bv: (b, 0)),
    )
    return pl.pallas_call(
        _mlp_body,
        grid_spec=grid_spec,
        out_shape=jax.ShapeDtypeStruct((P, D), jnp.float32),
        compiler_params=pltpu.CompilerParams(
            dimension_semantics=("arbitrary", "arbitrary")),
    )(block_expert, block_valid, hs, W_in, b_in, W_out, b_out, rw2d)


def _schedule(route):
    e1 = route[:, 0].astype(jnp.int32)
    e2 = route[:, 1].astype(jnp.int32)
    w1 = route[:, 2]
    w2 = route[:, 3]
    e_flat = jnp.concatenate([e1, e2])
    w_flat = jnp.concatenate([w1, w2])
    onehot = (e_flat[:, None] == jnp.arange(E, dtype=jnp.int32)[None, :])
    csum = lax.associative_scan(jnp.add, onehot.astype(jnp.int32), axis=0)
    g = csum[-1]
    rank_in_e = jnp.take_along_axis(csum, e_flat[:, None], axis=1)[:, 0] - 1
    gp = ((g + BT - 1) // BT) * BT
    poff = jnp.concatenate([jnp.zeros((1,), jnp.int32), jnp.cumsum(gp)[:-1]])
    pp = rank_in_e + poff[e_flat]
    tok = jnp.arange(A, dtype=jnp.int32) % T
    row_token = (jnp.arange(P, dtype=jnp.int32) % T).at[pp].set(tok)
    row_w = jnp.zeros((P,), jnp.float32).at[pp].set(w_flat)
    total = jnp.sum(gp)
    bstart = jnp.arange(NBLK, dtype=jnp.int32) * BT
    block_expert = jnp.clip(
        jnp.searchsorted(poff, bstart, side="right").astype(jnp.int32) - 1,
        0, E - 1)
    block_valid = (bstart < total).astype(jnp.int32)
    return row_token, row_w, pp[:T], pp[T:], block_expert, block_valid


def kernel(x, gate_W, W_in, b_in, W_out, b_out):
    B, S, _ = x.shape
    x2d = x.reshape(T, D)
    gwt_pad = jnp.zeros((D, EPAD), jnp.float32).at[:, :E].set(gate_W.T)

    logits_p, route = _router(x2d, gwt_pad)
    router_logits = logits_p[:, :E]

    row_token, row_w, pos0, pos1, block_expert, block_valid = _schedule(route)

    hs = _make_sc_gather(P)(x2d, row_token)

    rw2d = jnp.broadcast_to(row_w[:, None], (P, 128))
    rows_out = _grouped_mlp(block_expert, block_valid, hs,
                            W_in, b_in.reshape(E * NF, 1, BF),
                            W_out, b_out.reshape(E, 1, D), rw2d)

    final2d = _make_sc_combine()(rows_out, pos0, pos1)
    return final2d.reshape(B, S, D), router_logits


# speedup vs baseline: 1.1523x; 1.1523x over previous
import functools

import jax
import jax.numpy as jnp
from jax import lax
from jax.experimental import pallas as pl
from jax.experimental.pallas import tpu as pltpu
from jax.experimental.pallas import tpu_sc as plsc

E = 8
K = 2
D = 1024
FF = 4096
T = 2048
A = T * K
BT = 512
NBLK = A // BT + E
P = NBLK * BT
BF = 1024
NF = FF // BF
EPAD = 128
L = 16


def _router_body(x_ref, gw_ref, logits_ref, route_ref):
    x = x_ref[...]
    gw = gw_ref[...]
    logits = jnp.dot(x, gw, preferred_element_type=jnp.float32)
    logits_ref[...] = logits
    col = lax.broadcasted_iota(jnp.int32, (T, EPAD), 1)
    valid = col < E
    ml = jnp.where(valid, logits, jnp.float32(-1e30))
    m = jnp.max(ml, axis=1, keepdims=True)
    ex = jnp.where(valid, jnp.exp(ml - m), 0.0)
    p = ex / jnp.sum(ex, axis=1, keepdims=True)
    w1 = jnp.max(p, axis=1, keepdims=True)
    e1 = jnp.min(jnp.where((p == w1) & valid, col, EPAD), axis=1, keepdims=True)
    p2 = jnp.where(valid & (col != e1), p, jnp.float32(-1.0))
    w2 = jnp.max(p2, axis=1, keepdims=True)
    e2 = jnp.min(jnp.where((p2 == w2) & valid, col, EPAD), axis=1, keepdims=True)
    s = w1 + w2
    w1n = w1 / s
    w2n = w2 / s
    route = jnp.where(col == 0, e1.astype(jnp.float32),
            jnp.where(col == 1, e2.astype(jnp.float32),
            jnp.where(col == 2, w1n,
            jnp.where(col == 3, w2n, 0.0))))
    route_ref[...] = route


def _router(x2d, gwt_pad):
    return pl.pallas_call(
        _router_body,
        out_shape=(jax.ShapeDtypeStruct((T, EPAD), jnp.float32),
                   jax.ShapeDtypeStruct((T, EPAD), jnp.float32)),
    )(x2d, gwt_pad)


def _prefix16(x):
    lane = lax.iota(jnp.int32, L)
    s = x
    for k in (1, 2, 4, 8):
        idx = jnp.maximum(lane - k, 0)
        sh = s.at[idx].get(mode="promise_in_bounds")
        s = s + jnp.clip(lane - (k - 1), 0, 1) * sh
    return s


def _make_sc_dispatch():
    info = plsc.get_sparse_core_info()
    nc = info.num_cores
    nw = nc * info.num_subcores
    tpw = T // nw
    mesh = plsc.VectorSubcoreMesh(core_axis_name="c", subcore_axis_name="s")

    @functools.partial(
        pl.kernel, mesh=mesh,
        out_type=(jax.ShapeDtypeStruct((P, D), jnp.float32),
                  jax.ShapeDtypeStruct((A,), jnp.int32),
                  jax.ShapeDtypeStruct((NBLK,), jnp.int32),
                  jax.ShapeDtypeStruct((NBLK,), jnp.int32)),
        scratch_types=[
            pltpu.VMEM((tpw,), jnp.float32),
            pltpu.VMEM((tpw,), jnp.float32),
            pltpu.VMEM((nw, L), jnp.int32),
            pltpu.VMEM((2, tpw), jnp.int32),
            pltpu.VMEM((tpw, D), jnp.float32),
            pltpu.VMEM((L,), jnp.int32),
            pltpu.VMEM((L,), jnp.int32),
            pltpu.SemaphoreType.DMA,
            pltpu.SemaphoreType.DMA,
        ],
    )
    def k(x_hbm, route4_hbm, counts_hbm, hs_hbm, pos_hbm, bexp_hbm, bval_hbm,
          e1_v, e2_v, cnt_v, dst_v, rows_v, st0_v, st1_v, sem0, sem1):
        wid = lax.axis_index("s") * nc + lax.axis_index("c")
        t0 = wid * tpw
        pltpu.sync_copy(route4_hbm.at[0, pl.ds(t0, tpw)], e1_v)
        pltpu.sync_copy(route4_hbm.at[1, pl.ds(t0, tpw)], e2_v)
        pltpu.sync_copy(counts_hbm, cnt_v)
        pltpu.sync_copy(x_hbm.at[pl.ds(t0, tpw)], rows_v)

        widv = jnp.full((L,), 0, jnp.int32) + wid
        zeros = jnp.zeros((L,), jnp.int32)
        g = zeros
        mine = zeros
        for r in range(nw):
            row = cnt_v[r, :]
            g = g + row
            rsplat = jnp.full((L,), r, jnp.int32)
            mine = mine + jnp.clip(widv - rsplat, 0, 1) * row
        lane = lax.iota(jnp.int32, L)
        gp = jnp.clip(E - lane, 0, 1) * (((g + (BT - 1)) >> 9) << 9)
        csum_gp = _prefix16(gp)
        poff = csum_gp - gp
        total = csum_gp[L - 1]
        bases = poff + mine

        base_s = [bases[e] for e in range(E)]
        for part in range(2):
            ev_ref = e1_v if part == 0 else e2_v
            for j in range(tpw // L):
                ev = ev_ref[pl.ds(j * L, L)].astype(jnp.int32)
                dst = zeros
                for e in range(E):
                    meq = 1 - jnp.clip(jnp.abs(ev - e), 0, 1)
                    pre = _prefix16(meq)
                    dst = dst + meq * (pre - 1 + base_s[e])
                    base_s[e] = base_s[e] + pre[L - 1]
                dst_v[part, pl.ds(j * L, L)] = dst

        pltpu.sync_copy(dst_v.at[0], pos_hbm.at[pl.ds(t0, tpw)])
        pltpu.sync_copy(dst_v.at[1], pos_hbm.at[pl.ds(T + t0, tpw)])
        cp0 = pltpu.async_copy(rows_v, hs_hbm.at[dst_v.at[0]], sem0)
        cp1 = pltpu.async_copy(rows_v, hs_hbm.at[dst_v.at[1]], sem1)

        @pl.when(wid == 0)
        def _():
            bstart = lax.iota(jnp.int32, L) * BT
            acc = zeros
            for e in range(E):
                pe = jnp.full((L,), 0, jnp.int32) + poff[e]
                acc = acc + jnp.clip(bstart - pe + 1, 0, 1)
            st0_v[...] = acc - 1
            st1_v[...] = jnp.clip(jnp.full((L,), 0, jnp.int32) + total - bstart,
                                  0, 1)
            pltpu.sync_copy(st0_v, bexp_hbm)
            pltpu.sync_copy(st1_v, bval_hbm)

        cp0.wait()
        cp1.wait()

    return k


def _make_sc_combine():
    info = plsc.get_sparse_core_info()
    nw = info.num_cores * info.num_subcores
    t_per_w = T // nw
    ch = 32
    nch = t_per_w // ch
    mesh = plsc.VectorSubcoreMesh(core_axis_name="c", subcore_axis_name="s")

    @functools.partial(
        pl.kernel, mesh=mesh,
        out_type=jax.ShapeDtypeStruct((T, D), jnp.float32),
        scratch_types=[
            pltpu.VMEM((t_per_w,), jnp.int32),
            pltpu.VMEM((t_per_w,), jnp.int32),
            pltpu.VMEM((ch, D), jnp.float32),
            pltpu.VMEM((ch, D), jnp.float32),
            pltpu.SemaphoreType.DMA,
        ],
    )
    def k(table_hbm, p0_hbm, p1_hbm, out_hbm, i0_v, i1_v, r0_v, r1_v, sem):
        wid = lax.axis_index("s") * info.num_cores + lax.axis_index("c")
        base = wid * t_per_w
        pltpu.sync_copy(p0_hbm.at[pl.ds(base, t_per_w)], i0_v)
        pltpu.sync_copy(p1_hbm.at[pl.ds(base, t_per_w)], i1_v)
        for c in range(nch):
            pltpu.async_copy(
                table_hbm.at[i0_v.at[pl.ds(c * ch, ch)]], r0_v, sem).wait()
            pltpu.async_copy(
                table_hbm.at[i1_v.at[pl.ds(c * ch, ch)]], r1_v, sem).wait()

            def body(i, carry):
                for j in range(D // 16):
                    sl = pl.ds(j * 16, 16)
                    r0_v[i, sl] = r0_v[i, sl] + r1_v[i, sl]
                return carry

            lax.fori_loop(0, ch, body, 0)
            pltpu.sync_copy(r0_v, out_hbm.at[pl.ds(base + c * ch, ch)])

    return k


def _mlp_body(be_ref, bv_ref, x_ref, win_ref, bin_ref, wout_ref, bout_ref,
              rw_ref, out_ref):
    f = pl.program_id(1)
    b = pl.program_id(0)

    @pl.when(bv_ref[b] == 1)
    def _():
        x = x_ref[...].astype(jnp.bfloat16)
        win = win_ref[0].astype(jnp.bfloat16)
        h = jnp.dot(x, win, preferred_element_type=jnp.float32)
        h = h + bin_ref[0]
        a = jax.nn.gelu(h).astype(jnp.bfloat16)
        wout = wout_ref[0].astype(jnp.bfloat16)
        contrib = jnp.dot(a, wout, preferred_element_type=jnp.float32)

        @pl.when(f == 0)
        def _():
            out_ref[...] = contrib + bout_ref[0]

        @pl.when(f != 0)
        def _():
            out_ref[...] = out_ref[...] + contrib

        @pl.when(f == NF - 1)
        def _():
            w = rw_ref[...][:, 0:1]
            out_ref[...] = out_ref[...] * w


def _grouped_mlp(block_expert, block_valid, hs, W_in, b_in, W_out, b_out, rw2d):
    grid_spec = pltpu.PrefetchScalarGridSpec(
        num_scalar_prefetch=2,
        grid=(NBLK, NF),
        in_specs=[
            pl.BlockSpec((BT, D), lambda b, f, be, bv: (b, 0)),
            pl.BlockSpec((1, D, BF), lambda b, f, be, bv: (be[b], 0, f)),
            pl.BlockSpec((1, 1, BF), lambda b, f, be, bv: (be[b] * NF + f, 0, 0)),
            pl.BlockSpec((1, BF, D), lambda b, f, be, bv: (be[b], f, 0)),
            pl.BlockSpec((1, 1, D), lambda b, f, be, bv: (be[b], 0, 0)),
            pl.BlockSpec((BT, 128), lambda b, f, be, bv: (b, 0)),
        ],
        out_specs=pl.BlockSpec((BT, D), lambda b, f, be, bv: (b, 0)),
    )
    return pl.pallas_call(
        _mlp_body,
        grid_spec=grid_spec,
        out_shape=jax.ShapeDtypeStruct((P, D), jnp.float32),
        compiler_params=pltpu.CompilerParams(
            dimension_semantics=("arbitrary", "arbitrary")),
    )(block_expert, block_valid, hs, W_in, b_in, W_out, b_out, rw2d)


def kernel(x, gate_W, W_in, b_in, W_out, b_out):
    B, S, _ = x.shape
    x2d = x.reshape(T, D)
    gwt_pad = jnp.zeros((D, EPAD), jnp.float32).at[:, :E].set(gate_W.T)

    logits_p, route = _router(x2d, gwt_pad)
    router_logits = logits_p[:, :E]

    route4 = route[:, :4].T
    e_flat = route4[:2].reshape(A).astype(jnp.int32)
    onehot = (e_flat[:, None] == jnp.arange(L, dtype=jnp.int32)[None, :])
    counts = onehot.astype(jnp.int32).reshape(2, T // 64, 64, L).sum(
        axis=(0, 2), dtype=jnp.int32)

    hs, pos, block_expert, block_valid = _make_sc_dispatch()(
        x2d, route4, counts)

    row_w = jnp.zeros((P,), jnp.float32).at[pos].set(route4[2:4].reshape(A))
    rw2d = jnp.broadcast_to(row_w[:, None], (P, 128))
    rows_out = _grouped_mlp(block_expert, block_valid, hs,
                            W_in, b_in.reshape(E * NF, 1, BF),
                            W_out, b_out.reshape(E, 1, D), rw2d)

    final2d = _make_sc_combine()(rows_out, pos[:T], pos[T:])
    return final2d.reshape(B, S, D), router_logits


# speedup vs baseline: 1.1569x; 1.0040x over previous
import functools

import jax
import jax.numpy as jnp
from jax import lax
from jax.experimental import pallas as pl
from jax.experimental.pallas import tpu as pltpu
from jax.experimental.pallas import tpu_sc as plsc

E = 8
K = 2
D = 1024
FF = 4096
T = 2048
A = T * K
BT = 512
NBLK = A // BT + E
P = NBLK * BT
BF = 1024
NF = FF // BF
EPAD = 128
L = 16


def _router_body(x_ref, gw_ref, logits_ref, route_ref):
    x = x_ref[...]
    gw = gw_ref[...]
    logits = jnp.dot(x, gw, preferred_element_type=jnp.float32)
    logits_ref[...] = logits
    col = lax.broadcasted_iota(jnp.int32, (T, EPAD), 1)
    valid = col < E
    ml = jnp.where(valid, logits, jnp.float32(-1e30))
    m = jnp.max(ml, axis=1, keepdims=True)
    ex = jnp.where(valid, jnp.exp(ml - m), 0.0)
    p = ex / jnp.sum(ex, axis=1, keepdims=True)
    w1 = jnp.max(p, axis=1, keepdims=True)
    e1 = jnp.min(jnp.where((p == w1) & valid, col, EPAD), axis=1, keepdims=True)
    p2 = jnp.where(valid & (col != e1), p, jnp.float32(-1.0))
    w2 = jnp.max(p2, axis=1, keepdims=True)
    e2 = jnp.min(jnp.where((p2 == w2) & valid, col, EPAD), axis=1, keepdims=True)
    s = w1 + w2
    w1n = w1 / s
    w2n = w2 / s
    route = jnp.where(col == 0, e1.astype(jnp.float32),
            jnp.where(col == 1, e2.astype(jnp.float32),
            jnp.where(col == 2, w1n,
            jnp.where(col == 3, w2n, 0.0))))
    route_ref[...] = route


def _router(x2d, gwt_pad):
    return pl.pallas_call(
        _router_body,
        out_shape=(jax.ShapeDtypeStruct((T, EPAD), jnp.float32),
                   jax.ShapeDtypeStruct((T, EPAD), jnp.float32)),
    )(x2d, gwt_pad)


def _prefix16(x):
    lane = lax.iota(jnp.int32, L)
    s = x
    for k in (1, 2, 4, 8):
        idx = jnp.maximum(lane - k, 0)
        sh = s.at[idx].get(mode="promise_in_bounds")
        s = s + jnp.clip(lane - (k - 1), 0, 1) * sh
    return s


def _make_sc_dispatch():
    info = plsc.get_sparse_core_info()
    nc = info.num_cores
    nw = nc * info.num_subcores
    tpw = T // nw
    mesh = plsc.VectorSubcoreMesh(core_axis_name="c", subcore_axis_name="s")

    @functools.partial(
        pl.kernel, mesh=mesh,
        out_type=(jax.ShapeDtypeStruct((P, D), jnp.float32),
                  jax.ShapeDtypeStruct((A,), jnp.int32),
                  jax.ShapeDtypeStruct((NBLK,), jnp.int32),
                  jax.ShapeDtypeStruct((NBLK,), jnp.int32)),
        scratch_types=[
            pltpu.VMEM((tpw,), jnp.float32),
            pltpu.VMEM((tpw,), jnp.float32),
            pltpu.VMEM((nw, L), jnp.int32),
            pltpu.VMEM((2, tpw), jnp.int32),
            pltpu.VMEM((tpw, D), jnp.float32),
            pltpu.VMEM((L,), jnp.int32),
            pltpu.VMEM((L,), jnp.int32),
            pltpu.SemaphoreType.DMA,
            pltpu.SemaphoreType.DMA,
        ],
    )
    def k(x_hbm, route4_hbm, counts_hbm, hs_hbm, pos_hbm, bexp_hbm, bval_hbm,
          e1_v, e2_v, cnt_v, dst_v, rows_v, st0_v, st1_v, sem0, sem1):
        wid = lax.axis_index("s") * nc + lax.axis_index("c")
        t0 = wid * tpw
        pltpu.sync_copy(route4_hbm.at[0, pl.ds(t0, tpw)], e1_v)
        pltpu.sync_copy(route4_hbm.at[1, pl.ds(t0, tpw)], e2_v)
        pltpu.sync_copy(counts_hbm, cnt_v)
        pltpu.sync_copy(x_hbm.at[pl.ds(t0, tpw)], rows_v)

        widv = jnp.full((L,), 0, jnp.int32) + wid
        zeros = jnp.zeros((L,), jnp.int32)
        g = zeros
        mine = zeros
        for r in range(nw):
            row = cnt_v[r, :]
            g = g + row
            rsplat = jnp.full((L,), r, jnp.int32)
            mine = mine + jnp.clip(widv - rsplat, 0, 1) * row
        lane = lax.iota(jnp.int32, L)
        gp = jnp.clip(E - lane, 0, 1) * (((g + (BT - 1)) >> 9) << 9)
        csum_gp = _prefix16(gp)
        poff = csum_gp - gp
        total = csum_gp[L - 1]
        bases = poff + mine

        base_s = [bases[e] for e in range(E)]
        for part in range(2):
            ev_ref = e1_v if part == 0 else e2_v
            for j in range(tpw // L):
                ev = ev_ref[pl.ds(j * L, L)].astype(jnp.int32)
                dst = zeros
                for e in range(E):
                    meq = 1 - jnp.clip(jnp.abs(ev - e), 0, 1)
                    pre = _prefix16(meq)
                    dst = dst + meq * (pre - 1 + base_s[e])
                    base_s[e] = base_s[e] + pre[L - 1]
                dst_v[part, pl.ds(j * L, L)] = dst

        pltpu.sync_copy(dst_v.at[0], pos_hbm.at[pl.ds(t0, tpw)])
        pltpu.sync_copy(dst_v.at[1], pos_hbm.at[pl.ds(T + t0, tpw)])
        cp0 = pltpu.async_copy(rows_v, hs_hbm.at[dst_v.at[0]], sem0)
        cp1 = pltpu.async_copy(rows_v, hs_hbm.at[dst_v.at[1]], sem1)

        @pl.when(wid == 0)
        def _():
            bstart = lax.iota(jnp.int32, L) * BT
            acc = zeros
            for e in range(E):
                pe = jnp.full((L,), 0, jnp.int32) + poff[e]
                acc = acc + jnp.clip(bstart - pe + 1, 0, 1)
            st0_v[...] = acc - 1
            st1_v[...] = jnp.clip(jnp.full((L,), 0, jnp.int32) + total - bstart,
                                  0, 1)
            pltpu.sync_copy(st0_v, bexp_hbm)
            pltpu.sync_copy(st1_v, bval_hbm)

        cp0.wait()
        cp1.wait()

    return k


def _make_sc_combine():
    info = plsc.get_sparse_core_info()
    nw = info.num_cores * info.num_subcores
    t_per_w = T // nw
    ch = 32
    nch = t_per_w // ch
    mesh = plsc.VectorSubcoreMesh(core_axis_name="c", subcore_axis_name="s")

    @functools.partial(
        pl.kernel, mesh=mesh,
        out_type=jax.ShapeDtypeStruct((T, D), jnp.float32),
        scratch_types=[
            pltpu.VMEM((t_per_w,), jnp.int32),
            pltpu.VMEM((t_per_w,), jnp.int32),
            pltpu.VMEM((ch, D), jnp.float32),
            pltpu.VMEM((ch, D), jnp.float32),
            pltpu.VMEM((ch, D), jnp.float32),
            pltpu.SemaphoreType.DMA,
            pltpu.SemaphoreType.DMA,
            pltpu.SemaphoreType.DMA,
            pltpu.SemaphoreType.DMA,
        ],
    )
    def k(table_hbm, p0_hbm, p1_hbm, out_hbm, i0_v, i1_v, ra_v, rb_v, r1_v,
          s0, s1, swa, swb):
        wid = lax.axis_index("s") * info.num_cores + lax.axis_index("c")
        base = wid * t_per_w
        pltpu.sync_copy(p0_hbm.at[pl.ds(base, t_per_w)], i0_v)
        pltpu.sync_copy(p1_hbm.at[pl.ds(base, t_per_w)], i1_v)
        accs = (ra_v, rb_v)
        wsems = (swa, swb)
        wrs = []
        for c in range(nch):
            acc = accs[c]
            cp0 = pltpu.async_copy(
                table_hbm.at[i0_v.at[pl.ds(c * ch, ch)]], acc, s0)
            cp1 = pltpu.async_copy(
                table_hbm.at[i1_v.at[pl.ds(c * ch, ch)]], r1_v, s1)
            cp0.wait()
            cp1.wait()

            def body(i, carry):
                for j in range(D // 16):
                    sl = pl.ds(j * 16, 16)
                    acc[i, sl] = acc[i, sl] + r1_v[i, sl]
                return carry

            lax.fori_loop(0, ch, body, 0)
            wrs.append(pltpu.async_copy(
                acc, out_hbm.at[pl.ds(base + c * ch, ch)], wsems[c]))
        for wr in wrs:
            wr.wait()

    return k


def _mlp_body(be_ref, bv_ref, x_ref, win_ref, bin_ref, wout_ref, bout_ref,
              rw_ref, out_ref):
    f = pl.program_id(1)
    b = pl.program_id(0)

    @pl.when(bv_ref[b] == 1)
    def _():
        x = x_ref[...].astype(jnp.bfloat16)
        win = win_ref[0].astype(jnp.bfloat16)
        h = jnp.dot(x, win, preferred_element_type=jnp.float32)
        h = h + bin_ref[0]
        a = jax.nn.gelu(h).astype(jnp.bfloat16)
        wout = wout_ref[0].astype(jnp.bfloat16)
        contrib = jnp.dot(a, wout, preferred_element_type=jnp.float32)

        @pl.when(f == 0)
        def _():
            out_ref[...] = contrib + bout_ref[0]

        @pl.when(f != 0)
        def _():
            out_ref[...] = out_ref[...] + contrib

        @pl.when(f == NF - 1)
        def _():
            w = rw_ref[...][:, 0:1]
            out_ref[...] = out_ref[...] * w


def _grouped_mlp(block_expert, block_valid, hs, W_in, b_in, W_out, b_out, rw2d):
    grid_spec = pltpu.PrefetchScalarGridSpec(
        num_scalar_prefetch=2,
        grid=(NBLK, NF),
        in_specs=[
            pl.BlockSpec((BT, D), lambda b, f, be, bv: (b, 0)),
            pl.BlockSpec((1, D, BF), lambda b, f, be, bv: (be[b], 0, f)),
            pl.BlockSpec((1, 1, BF), lambda b, f, be, bv: (be[b] * NF + f, 0, 0)),
            pl.BlockSpec((1, BF, D), lambda b, f, be, bv: (be[b], f, 0)),
            pl.BlockSpec((1, 1, D), lambda b, f, be, bv: (be[b], 0, 0)),
            pl.BlockSpec((BT, 8), lambda b, f, be, bv: (b, 0)),
        ],
        out_specs=pl.BlockSpec((BT, D), lambda b, f, be, bv: (b, 0)),
    )
    return pl.pallas_call(
        _mlp_body,
        grid_spec=grid_spec,
        out_shape=jax.ShapeDtypeStruct((P, D), jnp.float32),
        compiler_params=pltpu.CompilerParams(
            dimension_semantics=("arbitrary", "arbitrary")),
    )(block_expert, block_valid, hs, W_in, b_in, W_out, b_out, rw2d)


def kernel(x, gate_W, W_in, b_in, W_out, b_out):
    B, S, _ = x.shape
    x2d = x.reshape(T, D)
    gwt_pad = jnp.zeros((D, EPAD), jnp.float32).at[:, :E].set(gate_W.T)

    logits_p, route = _router(x2d, gwt_pad)
    router_logits = logits_p[:, :E]

    route4 = route[:, :4].T
    e_flat = route4[:2].reshape(A).astype(jnp.int32)
    onehot = (e_flat[:, None] == jnp.arange(L, dtype=jnp.int32)[None, :])
    counts = onehot.astype(jnp.int32).reshape(2, T // 64, 64, L).sum(
        axis=(0, 2), dtype=jnp.int32)

    hs, pos, block_expert, block_valid = _make_sc_dispatch()(
        x2d, route4, counts)

    row_w = jnp.zeros((P,), jnp.float32).at[pos].set(route4[2:4].reshape(A))
    rw2d = jnp.broadcast_to(row_w[:, None], (P, 8))
    rows_out = _grouped_mlp(block_expert, block_valid, hs,
                            W_in, b_in.reshape(E * NF, 1, BF),
                            W_out, b_out.reshape(E, 1, D), rw2d)

    final2d = _make_sc_combine()(rows_out, pos[:T], pos[T:])
    return final2d.reshape(B, S, D), router_logits


# speedup vs baseline: 1.2384x; 1.0705x over previous
import functools

import jax
import jax.numpy as jnp
from jax import lax
from jax.experimental import pallas as pl
from jax.experimental.pallas import tpu as pltpu
from jax.experimental.pallas import tpu_sc as plsc

E = 8
K = 2
D = 1024
FF = 4096
T = 2048
A = T * K
BT = 512
NBLK = A // BT + E
P = NBLK * BT
BF = 2048
NF = FF // BF
EPAD = 128
L = 16


def _router_body(x_ref, gw_ref, logits_ref, route_ref):
    x = x_ref[...]
    gw = gw_ref[...]
    logits = jnp.dot(x, gw, preferred_element_type=jnp.float32)
    logits_ref[...] = logits
    col = lax.broadcasted_iota(jnp.int32, (T, EPAD), 1)
    valid = col < E
    ml = jnp.where(valid, logits, jnp.float32(-1e30))
    m = jnp.max(ml, axis=1, keepdims=True)
    ex = jnp.where(valid, jnp.exp(ml - m), 0.0)
    p = ex / jnp.sum(ex, axis=1, keepdims=True)
    w1 = jnp.max(p, axis=1, keepdims=True)
    e1 = jnp.min(jnp.where((p == w1) & valid, col, EPAD), axis=1, keepdims=True)
    p2 = jnp.where(valid & (col != e1), p, jnp.float32(-1.0))
    w2 = jnp.max(p2, axis=1, keepdims=True)
    e2 = jnp.min(jnp.where((p2 == w2) & valid, col, EPAD), axis=1, keepdims=True)
    s = w1 + w2
    w1n = w1 / s
    w2n = w2 / s
    route = jnp.where(col == 0, e1.astype(jnp.float32),
            jnp.where(col == 1, e2.astype(jnp.float32),
            jnp.where(col == 2, w1n,
            jnp.where(col == 3, w2n, 0.0))))
    route_ref[...] = route


def _router(x2d, gwt_pad):
    return pl.pallas_call(
        _router_body,
        out_shape=(jax.ShapeDtypeStruct((T, EPAD), jnp.float32),
                   jax.ShapeDtypeStruct((T, EPAD), jnp.float32)),
    )(x2d, gwt_pad)


def _prefix16(x):
    lane = lax.iota(jnp.int32, L)
    s = x
    for k in (1, 2, 4, 8):
        idx = jnp.maximum(lane - k, 0)
        sh = s.at[idx].get(mode="promise_in_bounds")
        s = s + jnp.clip(lane - (k - 1), 0, 1) * sh
    return s


def _make_sc_dispatch():
    info = plsc.get_sparse_core_info()
    nc = info.num_cores
    nw = nc * info.num_subcores
    tpw = T // nw
    mesh = plsc.VectorSubcoreMesh(core_axis_name="c", subcore_axis_name="s")

    @functools.partial(
        pl.kernel, mesh=mesh,
        out_type=(jax.ShapeDtypeStruct((P, D), jnp.float32),
                  jax.ShapeDtypeStruct((A,), jnp.int32),
                  jax.ShapeDtypeStruct((NBLK,), jnp.int32),
                  jax.ShapeDtypeStruct((NBLK,), jnp.int32)),
        scratch_types=[
            pltpu.VMEM((tpw,), jnp.float32),
            pltpu.VMEM((tpw,), jnp.float32),
            pltpu.VMEM((nw, L), jnp.int32),
            pltpu.VMEM((2, tpw), jnp.int32),
            pltpu.VMEM((tpw, D), jnp.float32),
            pltpu.VMEM((L,), jnp.int32),
            pltpu.VMEM((L,), jnp.int32),
            pltpu.SemaphoreType.DMA,
            pltpu.SemaphoreType.DMA,
        ],
    )
    def k(x_hbm, route4_hbm, counts_hbm, hs_hbm, pos_hbm, bexp_hbm, bval_hbm,
          e1_v, e2_v, cnt_v, dst_v, rows_v, st0_v, st1_v, sem0, sem1):
        wid = lax.axis_index("s") * nc + lax.axis_index("c")
        t0 = wid * tpw
        pltpu.sync_copy(route4_hbm.at[0, pl.ds(t0, tpw)], e1_v)
        pltpu.sync_copy(route4_hbm.at[1, pl.ds(t0, tpw)], e2_v)
        pltpu.sync_copy(counts_hbm, cnt_v)
        pltpu.sync_copy(x_hbm.at[pl.ds(t0, tpw)], rows_v)

        widv = jnp.full((L,), 0, jnp.int32) + wid
        zeros = jnp.zeros((L,), jnp.int32)
        g = zeros
        mine = zeros
        for r in range(nw):
            row = cnt_v[r, :]
            g = g + row
            rsplat = jnp.full((L,), r, jnp.int32)
            mine = mine + jnp.clip(widv - rsplat, 0, 1) * row
        lane = lax.iota(jnp.int32, L)
        gp = jnp.clip(E - lane, 0, 1) * (((g + (BT - 1)) >> 9) << 9)
        csum_gp = _prefix16(gp)
        poff = csum_gp - gp
        total = csum_gp[L - 1]
        bases = poff + mine

        base_s = [bases[e] for e in range(E)]
        for part in range(2):
            ev_ref = e1_v if part == 0 else e2_v
            for j in range(tpw // L):
                ev = ev_ref[pl.ds(j * L, L)].astype(jnp.int32)
                dst = zeros
                for e in range(E):
                    meq = 1 - jnp.clip(jnp.abs(ev - e), 0, 1)
                    pre = _prefix16(meq)
                    dst = dst + meq * (pre - 1 + base_s[e])
                    base_s[e] = base_s[e] + pre[L - 1]
                dst_v[part, pl.ds(j * L, L)] = dst

        pltpu.sync_copy(dst_v.at[0], pos_hbm.at[pl.ds(t0, tpw)])
        pltpu.sync_copy(dst_v.at[1], pos_hbm.at[pl.ds(T + t0, tpw)])
        cp0 = pltpu.async_copy(rows_v, hs_hbm.at[dst_v.at[0]], sem0)
        cp1 = pltpu.async_copy(rows_v, hs_hbm.at[dst_v.at[1]], sem1)

        @pl.when(wid == 0)
        def _():
            bstart = lax.iota(jnp.int32, L) * BT
            acc = zeros
            for e in range(E):
                pe = jnp.full((L,), 0, jnp.int32) + poff[e]
                acc = acc + jnp.clip(bstart - pe + 1, 0, 1)
            st0_v[...] = acc - 1
            st1_v[...] = jnp.clip(jnp.full((L,), 0, jnp.int32) + total - bstart,
                                  0, 1)
            pltpu.sync_copy(st0_v, bexp_hbm)
            pltpu.sync_copy(st1_v, bval_hbm)

        cp0.wait()
        cp1.wait()

    return k


def _make_sc_combine():
    info = plsc.get_sparse_core_info()
    nw = info.num_cores * info.num_subcores
    t_per_w = T // nw
    ch = 32
    nch = t_per_w // ch
    mesh = plsc.VectorSubcoreMesh(core_axis_name="c", subcore_axis_name="s")

    @functools.partial(
        pl.kernel, mesh=mesh,
        out_type=jax.ShapeDtypeStruct((T, D), jnp.float32),
        scratch_types=[
            pltpu.VMEM((t_per_w,), jnp.int32),
            pltpu.VMEM((t_per_w,), jnp.int32),
            pltpu.VMEM((ch, D), jnp.float32),
            pltpu.VMEM((ch, D), jnp.float32),
            pltpu.VMEM((ch, D), jnp.float32),
            pltpu.SemaphoreType.DMA,
            pltpu.SemaphoreType.DMA,
            pltpu.SemaphoreType.DMA,
            pltpu.SemaphoreType.DMA,
        ],
    )
    def k(table_hbm, p0_hbm, p1_hbm, out_hbm, i0_v, i1_v, ra_v, rb_v, r1_v,
          s0, s1, swa, swb):
        wid = lax.axis_index("s") * info.num_cores + lax.axis_index("c")
        base = wid * t_per_w
        pltpu.sync_copy(p0_hbm.at[pl.ds(base, t_per_w)], i0_v)
        pltpu.sync_copy(p1_hbm.at[pl.ds(base, t_per_w)], i1_v)
        accs = (ra_v, rb_v)
        wsems = (swa, swb)
        wrs = []
        for c in range(nch):
            acc = accs[c]
            cp0 = pltpu.async_copy(
                table_hbm.at[i0_v.at[pl.ds(c * ch, ch)]], acc, s0)
            cp1 = pltpu.async_copy(
                table_hbm.at[i1_v.at[pl.ds(c * ch, ch)]], r1_v, s1)
            cp0.wait()
            cp1.wait()

            def body(i, carry):
                for j in range(D // 16):
                    sl = pl.ds(j * 16, 16)
                    acc[i, sl] = acc[i, sl] + r1_v[i, sl]
                return carry

            lax.fori_loop(0, ch, body, 0)
            wrs.append(pltpu.async_copy(
                acc, out_hbm.at[pl.ds(base + c * ch, ch)], wsems[c]))
        for wr in wrs:
            wr.wait()

    return k


def _mlp_body(be_ref, bv_ref, x_ref, win_ref, bin_ref, wout_ref, bout_ref,
              rw_ref, out_ref):
    f = pl.program_id(1)
    b = pl.program_id(0)

    @pl.when(bv_ref[b] == 1)
    def _():
        x = x_ref[...].astype(jnp.bfloat16)
        win = win_ref[0].astype(jnp.bfloat16)
        h = jnp.dot(x, win, preferred_element_type=jnp.float32)
        h = h + bin_ref[0]
        a = jax.nn.gelu(h).astype(jnp.bfloat16)
        wout = wout_ref[0].astype(jnp.bfloat16)
        contrib = jnp.dot(a, wout, preferred_element_type=jnp.float32)

        @pl.when(f == 0)
        def _():
            out_ref[...] = contrib + bout_ref[0]

        @pl.when(f != 0)
        def _():
            out_ref[...] = out_ref[...] + contrib

        @pl.when(f == NF - 1)
        def _():
            w = rw_ref[...][:, 0:1]
            out_ref[...] = out_ref[...] * w


def _grouped_mlp(block_expert, block_valid, hs, W_in, b_in, W_out, b_out, rw2d):
    grid_spec = pltpu.PrefetchScalarGridSpec(
        num_scalar_prefetch=2,
        grid=(NBLK, NF),
        in_specs=[
            pl.BlockSpec((BT, D), lambda b, f, be, bv: (b, 0)),
            pl.BlockSpec((1, D, BF), lambda b, f, be, bv: (be[b], 0, f)),
            pl.BlockSpec((1, 1, BF), lambda b, f, be, bv: (be[b] * NF + f, 0, 0)),
            pl.BlockSpec((1, BF, D), lambda b, f, be, bv: (be[b], f, 0)),
            pl.BlockSpec((1, 1, D), lambda b, f, be, bv: (be[b], 0, 0)),
            pl.BlockSpec((BT, 8), lambda b, f, be, bv: (b, 0)),
        ],
        out_specs=pl.BlockSpec((BT, D), lambda b, f, be, bv: (b, 0)),
    )
    return pl.pallas_call(
        _mlp_body,
        grid_spec=grid_spec,
        out_shape=jax.ShapeDtypeStruct((P, D), jnp.float32),
        compiler_params=pltpu.CompilerParams(
            dimension_semantics=("arbitrary", "arbitrary")),
    )(block_expert, block_valid, hs, W_in, b_in, W_out, b_out, rw2d)


def kernel(x, gate_W, W_in, b_in, W_out, b_out):
    B, S, _ = x.shape
    x2d = x.reshape(T, D)
    gwt_pad = jnp.zeros((D, EPAD), jnp.float32).at[:, :E].set(gate_W.T)

    logits_p, route = _router(x2d, gwt_pad)
    router_logits = logits_p[:, :E]

    route4 = route[:, :4].T
    e_flat = route4[:2].reshape(A).astype(jnp.int32)
    onehot = (e_flat[:, None] == jnp.arange(L, dtype=jnp.int32)[None, :])
    counts = onehot.astype(jnp.int32).reshape(2, T // 64, 64, L).sum(
        axis=(0, 2), dtype=jnp.int32)

    hs, pos, block_expert, block_valid = _make_sc_dispatch()(
        x2d, route4, counts)

    row_w = jnp.zeros((P,), jnp.float32).at[pos].set(route4[2:4].reshape(A))
    rw2d = jnp.broadcast_to(row_w[:, None], (P, 8))
    rows_out = _grouped_mlp(block_expert, block_valid, hs,
                            W_in, b_in.reshape(E * NF, 1, BF),
                            W_out, b_out.reshape(E, 1, D), rw2d)

    final2d = _make_sc_combine()(rows_out, pos[:T], pos[T:])
    return final2d.reshape(B, S, D), router_logits
